# baseline (speedup 1.0000x reference)
import jax
import jax.numpy as jnp
from jax.experimental import pallas as pl
from jax.experimental.pallas import tpu as pltpu

_PREC = jax.lax.Precision.HIGHEST


def _embed_kernel(x_ref,
                  wl1_ref, b1g_ref, wco1_ref,
                  wl2_ref, b2g_ref, wco2_ref,
                  wp_ref, b1_ref,
                  a_ref, bt_ref):
    h = x_ref[:]
    hid = a_ref.shape[1]
    for (w, bg, wco) in ((wl1_ref, b1g_ref, wco1_ref),
                         (wl2_ref, b2g_ref, wco2_ref)):
        g = jnp.dot(h, w[:], precision=_PREC,
                    preferred_element_type=jnp.float32) + bg[:]
        gi = jax.nn.sigmoid(g[:, :hid])
        gt = jnp.tanh(g[:, hid:2 * hid])
        c = gi * gt
        go = jax.nn.sigmoid(g[:, 2 * hid:] + wco[:] * c)
        h = go * jnp.tanh(c)

    p = jnp.dot(h, wp_ref[:], precision=_PREC,
                preferred_element_type=jnp.float32)
    a_ref[:] = p[:, :hid] + b1_ref[:]
    bt_ref[:] = p[:, hid:].T


def _score_kernel(a_ref, bt_ref, w2_ref, b2_ref, out_ref):
    bt = bt_ref[:]
    w2 = w2_ref[:]
    b2 = b2_ref[:]
    br = a_ref.shape[0]
    bc = bt.shape[1]
    hid = bt.shape[0]
    for i0 in range(0, br, 8):
        a8 = a_ref[i0:i0 + 8, :]
        acc = jnp.broadcast_to(b2, (8, bc))
        for k in range(hid):
            acc = acc + jnp.maximum(a8[:, k:k + 1] + bt[k:k + 1, :], 0.0) * w2[0:1, k:k + 1]
        out_ref[i0:i0 + 8, :] = jax.nn.sigmoid(acc)


def kernel(x, edge_weight, params, edge_index):
    del edge_weight, edge_index
    n = x.shape[0]
    hid = params["lp_w2"].shape[0]

    args = [x]
    for p in params["layers"]:
        args += [
            jnp.concatenate([p["W_i"], p["W_c"], p["W_o"]], axis=1),
            jnp.concatenate([
                (p["b_i"] + p["conv_i_b"][None, :]),
                (p["b_c"] + p["conv_c_b"][None, :]),
                (p["b_o"] + p["conv_o_b"][None, :]),
            ], axis=1).astype(jnp.float32),
            p["w_c_o"],
        ]
    args += [
        jnp.concatenate([params["lp_w1"][:hid], params["lp_w1"][hid:]], axis=1),
        params["lp_b1"][None, :],
    ]

    a, bt = pl.pallas_call(
        _embed_kernel,
        out_shape=(jax.ShapeDtypeStruct((n, hid), jnp.float32),
                   jax.ShapeDtypeStruct((hid, n), jnp.float32)),
    )(*args)

    br = 64
    return pl.pallas_call(
        _score_kernel,
        grid=(n // br,),
        in_specs=[
            pl.BlockSpec((br, hid), lambda i: (i, 0)),
            pl.BlockSpec((hid, n), lambda i: (0, 0)),
            pl.BlockSpec((1, hid), lambda i: (0, 0)),
            pl.BlockSpec((1, 1), lambda i: (0, 0)),
        ],
        out_specs=pl.BlockSpec((br, n), lambda i: (i, 0)),
        out_shape=jax.ShapeDtypeStruct((n, n), jnp.float32),
        compiler_params=pltpu.CompilerParams(
            dimension_semantics=("parallel",)),
    )(a, bt, params["lp_w2"].reshape(1, hid), params["lp_b2"].reshape(1, 1))


# speedup vs baseline: 1.0054x; 1.0054x over previous
import jax
import jax.numpy as jnp
from jax.experimental import pallas as pl
from jax.experimental.pallas import tpu as pltpu

_PREC = jax.lax.Precision.HIGHEST


def _embed_kernel(x_ref,
                  wl1_ref, b1g_ref, wco1_ref,
                  wl2_ref, b2g_ref, wco2_ref,
                  wp_ref, b1_ref,
                  a_ref, bt_ref):
    h = x_ref[:]
    hid = a_ref.shape[1]
    for (w, bg, wco) in ((wl1_ref, b1g_ref, wco1_ref),
                         (wl2_ref, b2g_ref, wco2_ref)):
        g = jnp.dot(h, w[:], precision=_PREC,
                    preferred_element_type=jnp.float32) + bg[:]
        gi = jax.nn.sigmoid(g[:, :hid])
        gt = jnp.tanh(g[:, hid:2 * hid])
        c = gi * gt
        go = jax.nn.sigmoid(g[:, 2 * hid:] + wco[:] * c)
        h = go * jnp.tanh(c)

    p = jnp.dot(h, wp_ref[:], precision=_PREC,
                preferred_element_type=jnp.float32)
    a_ref[:] = p[:, :hid] + b1_ref[:]
    bt_ref[:] = p[:, hid:].T


def _score_kernel(a_ref, bt_ref, w2_ref, b2_ref, out_ref):
    bt = bt_ref[:]
    w2 = w2_ref[:]
    b2 = b2_ref[:]
    br = a_ref.shape[0]
    bc = bt.shape[1]
    hid = bt.shape[0]
    for i0 in range(0, br, 16):
        a8 = a_ref[i0:i0 + 16, :]
        acc = jnp.broadcast_to(b2, (16, bc))
        for k in range(hid):
            acc = acc + jnp.maximum(a8[:, k:k + 1] + bt[k:k + 1, :], 0.0) * w2[0:1, k:k + 1]
        out_ref[i0:i0 + 16, :] = jax.nn.sigmoid(acc)


def kernel(x, edge_weight, params, edge_index):
    del edge_weight, edge_index
    n = x.shape[0]
    hid = params["lp_w2"].shape[0]

    args = [x]
    for p in params["layers"]:
        args += [
            jnp.concatenate([p["W_i"], p["W_c"], p["W_o"]], axis=1),
            jnp.concatenate([
                (p["b_i"] + p["conv_i_b"][None, :]),
                (p["b_c"] + p["conv_c_b"][None, :]),
                (p["b_o"] + p["conv_o_b"][None, :]),
            ], axis=1).astype(jnp.float32),
            p["w_c_o"],
        ]
    args += [
        jnp.concatenate([params["lp_w1"][:hid], params["lp_w1"][hid:]], axis=1),
        params["lp_b1"][None, :],
    ]

    a, bt = pl.pallas_call(
        _embed_kernel,
        out_shape=(jax.ShapeDtypeStruct((n, hid), jnp.float32),
                   jax.ShapeDtypeStruct((hid, n), jnp.float32)),
    )(*args)

    br = 128
    return pl.pallas_call(
        _score_kernel,
        grid=(n // br,),
        in_specs=[
            pl.BlockSpec((br, hid), lambda i: (i, 0)),
            pl.BlockSpec((hid, n), lambda i: (0, 0)),
            pl.BlockSpec((1, hid), lambda i: (0, 0)),
            pl.BlockSpec((1, 1), lambda i: (0, 0)),
        ],
        out_specs=pl.BlockSpec((br, n), lambda i: (i, 0)),
        out_shape=jax.ShapeDtypeStruct((n, n), jnp.float32),
        compiler_params=pltpu.CompilerParams(
            dimension_semantics=("parallel",)),
    )(a, bt, params["lp_w2"].reshape(1, hid), params["lp_b2"].reshape(1, 1))


# speedup vs baseline: 1.0435x; 1.0379x over previous
import jax
import jax.numpy as jnp
from jax.experimental import pallas as pl
from jax.experimental.pallas import tpu as pltpu

_PREC = jax.lax.Precision.HIGHEST


def _embed_kernel(x_ref,
                  wl1_ref, b1g_ref, wco1_ref,
                  wl2_ref, b2g_ref, wco2_ref,
                  wp_ref, b1_ref,
                  a_ref, bt_ref):
    h = x_ref[:]
    hid = a_ref.shape[1]
    for (w, bg, wco) in ((wl1_ref, b1g_ref, wco1_ref),
                         (wl2_ref, b2g_ref, wco2_ref)):
        g = jnp.dot(h, w[:], precision=_PREC,
                    preferred_element_type=jnp.float32) + bg[:]
        gi = jax.nn.sigmoid(g[:, :hid])
        gt = jnp.tanh(g[:, hid:2 * hid])
        c = gi * gt
        go = jax.nn.sigmoid(g[:, 2 * hid:] + wco[:] * c)
        h = go * jnp.tanh(c)

    p = jnp.dot(h, wp_ref[:], precision=_PREC,
                preferred_element_type=jnp.float32)
    a_ref[:] = p[:, :hid] + b1_ref[:]
    bt_ref[:] = p[:, hid:].T


def _score_kernel(a_ref, bt_ref, w2_ref, b2_ref, out_ref):
    bt = bt_ref[:]
    w2 = w2_ref[:]
    b2 = b2_ref[:]
    br = a_ref.shape[0]
    bc = bt.shape[1]
    hid = bt.shape[0]
    for i0 in range(0, br, 8):
        a8 = a_ref[i0:i0 + 8, :]
        acc = jnp.broadcast_to(b2, (8, bc))
        for k in range(hid):
            acc = acc + jnp.maximum(a8[:, k:k + 1] + bt[k:k + 1, :], 0.0) * w2[0:1, k:k + 1]
        out_ref[i0:i0 + 8, :] = jax.nn.sigmoid(acc)


def kernel(x, edge_weight, params, edge_index):
    del edge_weight, edge_index
    n = x.shape[0]
    hid = params["lp_w2"].shape[0]

    args = [x]
    for p in params["layers"]:
        args += [
            jnp.concatenate([p["W_i"], p["W_c"], p["W_o"]], axis=1),
            jnp.concatenate([
                (p["b_i"] + p["conv_i_b"][None, :]),
                (p["b_c"] + p["conv_c_b"][None, :]),
                (p["b_o"] + p["conv_o_b"][None, :]),
            ], axis=1).astype(jnp.float32),
            p["w_c_o"],
        ]
    args += [
        jnp.concatenate([params["lp_w1"][:hid], params["lp_w1"][hid:]], axis=1),
        params["lp_b1"][None, :],
    ]

    a, bt = pl.pallas_call(
        _embed_kernel,
        out_shape=(jax.ShapeDtypeStruct((n, hid), jnp.float32),
                   jax.ShapeDtypeStruct((hid, n), jnp.float32)),
    )(*args)

    br = 256
    return pl.pallas_call(
        _score_kernel,
        grid=(n // br,),
        in_specs=[
            pl.BlockSpec((br, hid), lambda i: (i, 0)),
            pl.BlockSpec((hid, n), lambda i: (0, 0)),
            pl.BlockSpec((1, hid), lambda i: (0, 0)),
            pl.BlockSpec((1, 1), lambda i: (0, 0)),
        ],
        out_specs=pl.BlockSpec((br, n), lambda i: (i, 0)),
        out_shape=jax.ShapeDtypeStruct((n, n), jnp.float32),
        compiler_params=pltpu.CompilerParams(
            dimension_semantics=("parallel",)),
    )(a, bt, params["lp_w2"].reshape(1, hid), params["lp_b2"].reshape(1, 1))


# speedup vs baseline: 1.0446x; 1.0011x over previous
import jax
import jax.numpy as jnp
from jax.experimental import pallas as pl
from jax.experimental.pallas import tpu as pltpu

_PREC = jax.lax.Precision.HIGHEST


def _embed_kernel(x_ref,
                  wl1_ref, b1g_ref, wco1_ref,
                  wl2_ref, b2g_ref, wco2_ref,
                  wp_ref, b1_ref,
                  a_ref, bt_ref):
    h = x_ref[:]
    hid = a_ref.shape[1]
    for (w, bg, wco) in ((wl1_ref, b1g_ref, wco1_ref),
                         (wl2_ref, b2g_ref, wco2_ref)):
        g = jnp.dot(h, w[:], precision=_PREC,
                    preferred_element_type=jnp.float32) + bg[:]
        gi = jax.nn.sigmoid(g[:, :hid])
        gt = jnp.tanh(g[:, hid:2 * hid])
        c = gi * gt
        go = jax.nn.sigmoid(g[:, 2 * hid:] + wco[:] * c)
        h = go * jnp.tanh(c)

    p = jnp.dot(h, wp_ref[:], precision=_PREC,
                preferred_element_type=jnp.float32)
    a_ref[:] = p[:, :hid] + b1_ref[:]
    bt_ref[:] = p[:, hid:].T


def _score_kernel(a_ref, bt_ref, w2_ref, b2_ref, out_ref):
    bt = bt_ref[:]
    w2 = w2_ref[:]
    b2 = b2_ref[:]
    br = a_ref.shape[0]
    bc = bt.shape[1]
    hid = bt.shape[0]
    for i0 in range(0, br, 8):
        a8 = a_ref[i0:i0 + 8, :]
        acc = jnp.broadcast_to(b2, (8, bc))
        for k in range(hid):
            acc = acc + jnp.maximum(a8[:, k:k + 1] + bt[k:k + 1, :], 0.0) * w2[0:1, k:k + 1]
        out_ref[i0:i0 + 8, :] = jax.nn.sigmoid(acc)


def kernel(x, edge_weight, params, edge_index):
    del edge_weight, edge_index
    n = x.shape[0]
    hid = params["lp_w2"].shape[0]

    args = [x]
    for p in params["layers"]:
        args += [
            jnp.concatenate([p["W_i"], p["W_c"], p["W_o"]], axis=1),
            jnp.concatenate([
                (p["b_i"] + p["conv_i_b"][None, :]),
                (p["b_c"] + p["conv_c_b"][None, :]),
                (p["b_o"] + p["conv_o_b"][None, :]),
            ], axis=1).astype(jnp.float32),
            p["w_c_o"],
        ]
    args += [
        jnp.concatenate([params["lp_w1"][:hid], params["lp_w1"][hid:]], axis=1),
        params["lp_b1"][None, :],
    ]

    a, bt = pl.pallas_call(
        _embed_kernel,
        out_shape=(jax.ShapeDtypeStruct((n, hid), jnp.float32),
                   jax.ShapeDtypeStruct((hid, n), jnp.float32)),
    )(*args)

    br = 512
    return pl.pallas_call(
        _score_kernel,
        grid=(n // br,),
        in_specs=[
            pl.BlockSpec((br, hid), lambda i: (i, 0)),
            pl.BlockSpec((hid, n), lambda i: (0, 0)),
            pl.BlockSpec((1, hid), lambda i: (0, 0)),
            pl.BlockSpec((1, 1), lambda i: (0, 0)),
        ],
        out_specs=pl.BlockSpec((br, n), lambda i: (i, 0)),
        out_shape=jax.ShapeDtypeStruct((n, n), jnp.float32),
        compiler_params=pltpu.CompilerParams(
            dimension_semantics=("parallel",)),
    )(a, bt, params["lp_w2"].reshape(1, hid), params["lp_b2"].reshape(1, 1))
